# Initial kernel scaffold; baseline (speedup 1.0000x reference)
import functools

import jax
import jax.numpy as jnp
from jax import lax
from jax.experimental import pallas as pl
from jax.experimental.pallas import tpu as pltpu
from jax.experimental.pallas import tpu_sc as plsc

L = 16
NC = 2
NS = 16
NW = NC * NS

ROWS = 16384
COLS = 200
D = 3
TAB_WORDS = 10 * D
BTOT = ROWS * COLS
BPW = BTOT // NW
CH = 6400
NCH = BPW // CH


def _make_kernel():
    mesh = plsc.VectorSubcoreMesh(core_axis_name="c", subcore_axis_name="s")

    @functools.partial(
        pl.kernel,
        out_type=jax.ShapeDtypeStruct((BTOT * D,), jnp.float32),
        mesh=mesh,
        scratch_types=[
            pltpu.VMEM((2 * L,), jnp.float32),
            pltpu.VMEM((CH,), jnp.int32),
            pltpu.VMEM((CH * D,), jnp.float32),
        ],
    )
    def emb_kernel(x_hbm, tab_hbm, out_hbm, tab_v, idx_v, out_v):
        wid = lax.axis_index("s") * NC + lax.axis_index("c")
        base = wid * BPW
        pltpu.sync_copy(tab_hbm, tab_v)
        iota3 = lax.broadcasted_iota(jnp.int32, (L,), 0) * D

        @pl.loop(0, NCH)
        def chunk_loop(c):
            off = base + c * CH
            pltpu.sync_copy(x_hbm.at[pl.ds(off, CH)], idx_v)

            @plsc.parallel_loop(0, CH // L, unroll=8)
            def group_loop(g):
                idx = idx_v[pl.ds(g * L, L)]
                b3 = idx * D
                pos = iota3 + g * (L * D)
                for kcol in range(D):
                    vals = plsc.load_gather(tab_v, [b3 + kcol])
                    plsc.store_scatter(out_v, [pos + kcol], vals)

            pltpu.sync_copy(out_v, out_hbm.at[pl.ds(off * D, CH * D)])

    return emb_kernel


_emb = _make_kernel()


def kernel(x, table):
    xf = x.reshape(-1)
    tab_pad = jnp.pad(table.reshape(-1), (0, 2 * L - TAB_WORDS))
    out = _emb(xf, tab_pad)
    return out.reshape(ROWS, COLS, D)


# speedup vs baseline: 5.7692x; 5.7692x over previous
import functools

import jax
import jax.numpy as jnp
from jax import lax
from jax.experimental import pallas as pl
from jax.experimental.pallas import tpu as pltpu
from jax.experimental.pallas import tpu_sc as plsc

L = 16
NC = 2
NS = 16
NW = NC * NS

ROWS = 16384
COLS = 200
D = 3
TAB_WORDS = 10 * D
RPW = ROWS // NW
CR = 64
NCH = RPW // CR
NG = COLS // L
TAIL = COLS - NG * L


def _make_kernel():
    mesh = plsc.VectorSubcoreMesh(core_axis_name="c", subcore_axis_name="s")

    @functools.partial(
        pl.kernel,
        out_type=jax.ShapeDtypeStruct((ROWS * COLS * D,), jnp.float32),
        mesh=mesh,
        scratch_types=[
            pltpu.VMEM((8 * L,), jnp.float32),
            pltpu.VMEM((CR, COLS), jnp.int32),
            pltpu.VMEM((CR * COLS * D,), jnp.float32),
        ],
        compiler_params=pltpu.CompilerParams(needs_layout_passes=False),
    )
    def emb_kernel(x_hbm, tab_hbm, out_hbm, tab_v, idx_v, out_v):
        wid = lax.axis_index("s") * NC + lax.axis_index("c")
        base_row = wid * RPW
        pltpu.sync_copy(tab_hbm, tab_v)
        iota3 = lax.broadcasted_iota(jnp.int32, (L,), 0) * D
        col_offs = [c * L for c in range(NG)] + [COLS - L]

        @pl.loop(0, NCH)
        def chunk_loop(ch):
            row0 = base_row + ch * CR
            pltpu.sync_copy(x_hbm.at[pl.ds(row0, CR), :], idx_v)

            @plsc.parallel_loop(0, CR, unroll=2)
            def row_loop(r):
                rbase = r * (COLS * D)
                for co in col_offs:
                    idx = idx_v[r, pl.ds(co, L)]
                    b3 = idx * D
                    pos = rbase + co * D + iota3
                    for kcol in range(D):
                        vals = plsc.load_gather(tab_v, [b3 + kcol])
                        plsc.store_scatter(out_v, [pos + kcol], vals)

            pltpu.sync_copy(out_v, out_hbm.at[pl.ds(row0 * COLS * D, CR * COLS * D)])

    return emb_kernel


_emb = _make_kernel()


def kernel(x, table):
    tab_pad = jnp.pad(table.reshape(-1), (0, 8 * L - TAB_WORDS))
    out = _emb(x, tab_pad)
    return out.reshape(ROWS, COLS, D)


# speedup vs baseline: 64.6874x; 11.2125x over previous
import functools

import jax
import jax.numpy as jnp
from jax import lax
from jax.experimental import pallas as pl
from jax.experimental.pallas import tpu as pltpu
from jax.experimental.pallas import tpu_sc as plsc

L = 16
NC = 2
NS = 16
NW = NC * NS

ROWS = 16384
COLS = 200
D = 3
VOCAB = 10
RPW = ROWS // NW
CI = 128
NCH = RPW // CI
J_OFFS = [j * L for j in range(COLS // L)] + [COLS - L]


def _make_kernel():
    mesh = plsc.VectorSubcoreMesh(core_axis_name="c", subcore_axis_name="s")

    @functools.partial(
        pl.kernel,
        out_type=jax.ShapeDtypeStruct((D, COLS, ROWS), jnp.float32),
        mesh=mesh,
        scratch_types=[
            pltpu.VMEM((D * L,), jnp.float32),
            pltpu.VMEM((CI, COLS), jnp.int32),
            [pltpu.VMEM((COLS, CI), jnp.float32) for _ in range(D)],
        ],
        compiler_params=pltpu.CompilerParams(needs_layout_passes=False),
    )
    def emb_kernel(x_hbm, tab_hbm, out_hbm, tab_v, xb, ob):
        wid = lax.axis_index("s") * NC + lax.axis_index("c")
        pltpu.sync_copy(tab_hbm, tab_v)
        iota = lax.broadcasted_iota(jnp.int32, (L,), 0)

        @pl.loop(0, NCH)
        def chunk_loop(ch):
            i0 = wid * RPW + ch * CI
            pltpu.sync_copy(x_hbm.at[pl.ds(i0, CI), :], xb)

            @plsc.parallel_loop(0, CI, unroll=2)
            def row_loop(ii):
                cols = jnp.broadcast_to(ii, (L,))
                for j0 in J_OFFS:
                    idx = xb[ii, pl.ds(j0, L)]
                    rows = iota + j0
                    for k in range(D):
                        vals = plsc.load_gather(tab_v, [idx + k * L])
                        plsc.store_scatter(ob[k], [rows, cols], vals)

            for k in range(D):
                pltpu.sync_copy(ob[k], out_hbm.at[k, :, pl.ds(i0, CI)])

    return emb_kernel


_emb = _make_kernel()


def kernel(x, table):
    tab_p = (
        jnp.zeros((D, L), jnp.float32).at[:, :VOCAB].set(table.T).reshape(D * L)
    )
    out = _emb(x, tab_p)
    return out.transpose(2, 1, 0)


# speedup vs baseline: 228.7113x; 3.5356x over previous
import functools

import jax
import jax.numpy as jnp
from jax import lax
from jax.experimental import pallas as pl
from jax.experimental.pallas import tpu as pltpu
from jax.experimental.pallas import tpu_sc as plsc

L = 16
NC = 2
NS = 16
NW = NC * NS

ROWS = 16384
COLS = 200
D = 3
VOCAB = 10
RPW = ROWS // NW
CI = 128
NCH = RPW // CI
NG = CI // L


def _make_kernel():
    mesh = plsc.VectorSubcoreMesh(core_axis_name="c", subcore_axis_name="s")

    @functools.partial(
        pl.kernel,
        out_type=jax.ShapeDtypeStruct((D, COLS, ROWS), jnp.float32),
        mesh=mesh,
        scratch_types=[
            pltpu.VMEM((D * L,), jnp.float32),
            pltpu.VMEM((COLS, CI), jnp.int32),
            [pltpu.VMEM((COLS, CI), jnp.float32) for _ in range(D)],
        ],
        compiler_params=pltpu.CompilerParams(needs_layout_passes=False),
    )
    def emb_kernel(xt_hbm, tab_hbm, out_hbm, tab_v, xb, ob):
        wid = lax.axis_index("s") * NC + lax.axis_index("c")
        pltpu.sync_copy(tab_hbm, tab_v)

        @pl.loop(0, NCH)
        def chunk_loop(ch):
            i0 = wid * RPW + ch * CI
            pltpu.sync_copy(xt_hbm.at[:, pl.ds(i0, CI)], xb)

            @plsc.parallel_loop(0, COLS, unroll=2)
            def row_loop(j):
                for g in range(NG):
                    sl = pl.ds(g * L, L)
                    idx = xb[j, sl]
                    for k in range(D):
                        vals = plsc.load_gather(tab_v, [idx + k * L])
                        ob[k][j, sl] = vals

            for k in range(D):
                pltpu.sync_copy(ob[k], out_hbm.at[k, :, pl.ds(i0, CI)])

    return emb_kernel


_emb = _make_kernel()


def kernel(x, table):
    tab_p = (
        jnp.zeros((D, L), jnp.float32).at[:, :VOCAB].set(table.T).reshape(D * L)
    )
    out = _emb(x.T, tab_p)
    return out.transpose(2, 1, 0)


# speedup vs baseline: 242.0433x; 1.0583x over previous
import functools

import jax
import jax.numpy as jnp
from jax import lax
from jax.experimental import pallas as pl
from jax.experimental.pallas import tpu as pltpu
from jax.experimental.pallas import tpu_sc as plsc

L = 16
NC = 2
NS = 16
NW = NC * NS

ROWS = 16384
COLS = 200
D = 3
VOCAB = 10
RPW = ROWS // NW
CI = 128
NCH = RPW // CI
NG = CI // L
SLABS = ((0, 48), (48, 48), (96, 48), (144, 56))


def _make_kernel():
    mesh = plsc.VectorSubcoreMesh(core_axis_name="c", subcore_axis_name="s")

    @functools.partial(
        pl.kernel,
        out_type=jax.ShapeDtypeStruct((D, COLS, ROWS), jnp.float32),
        mesh=mesh,
        scratch_types=[
            pltpu.VMEM((D * L,), jnp.float32),
            [pltpu.VMEM((COLS, CI), jnp.int32) for _ in range(2)],
            [pltpu.VMEM((COLS, CI), jnp.float32) for _ in range(D)],
            [pltpu.SemaphoreType.DMA for _ in range(2)],
            [pltpu.SemaphoreType.DMA for _ in range(len(SLABS))],
        ],
        compiler_params=pltpu.CompilerParams(needs_layout_passes=False),
    )
    def emb_kernel(xt_hbm, tab_hbm, out_hbm, tab_v, xb, ob, in_sem, out_sem):
        wid = lax.axis_index("s") * NC + lax.axis_index("c")
        base = wid * RPW
        pltpu.sync_copy(tab_hbm, tab_v)

        def in_slice(ch):
            return xt_hbm.at[:, pl.ds(base + ch * CI, CI)]

        def out_slab(ch, k, s):
            j0, jn = SLABS[s]
            return out_hbm.at[k, pl.ds(j0, jn), pl.ds(base + ch * CI, CI)]

        def ob_slab(k, s):
            j0, jn = SLABS[s]
            return ob[k].at[pl.ds(j0, jn), :]

        def compute_slab(b, s):
            j0, jn = SLABS[s]

            @plsc.parallel_loop(j0, j0 + jn, unroll=2)
            def row_loop(j):
                for g in range(NG):
                    sl = pl.ds(g * L, L)
                    idx = xb[b][j, sl]
                    for k in range(D):
                        vals = plsc.load_gather(tab_v, [idx + k * L])
                        ob[k][j, sl] = vals

        pltpu.async_copy(in_slice(0), xb[0], in_sem[0])
        if NCH > 1:
            pltpu.async_copy(in_slice(1), xb[1], in_sem[1])
        for ch in range(NCH):
            b = ch % 2
            pltpu.make_async_copy(in_slice(ch), xb[b], in_sem[b]).wait()
            for s in range(len(SLABS)):
                if ch >= 1:
                    for k in range(D):
                        pltpu.make_async_copy(
                            ob_slab(k, s), out_slab(ch - 1, k, s), out_sem[s]
                        ).wait()
                compute_slab(b, s)
                for k in range(D):
                    pltpu.async_copy(ob_slab(k, s), out_slab(ch, k, s), out_sem[s])
            if ch + 2 < NCH:
                pltpu.async_copy(in_slice(ch + 2), xb[b], in_sem[b])
        for s in range(len(SLABS)):
            for k in range(D):
                pltpu.make_async_copy(
                    ob_slab(k, s), out_slab(NCH - 1, k, s), out_sem[s]
                ).wait()

    return emb_kernel


_emb = _make_kernel()


def kernel(x, table):
    tab_p = (
        jnp.zeros((D, L), jnp.float32).at[:, :VOCAB].set(table.T).reshape(D * L)
    )
    out = _emb(x.T, tab_p)
    return out.transpose(2, 1, 0)


# speedup vs baseline: 243.3660x; 1.0055x over previous
import functools

import jax
import jax.numpy as jnp
from jax import lax
from jax.experimental import pallas as pl
from jax.experimental.pallas import tpu as pltpu
from jax.experimental.pallas import tpu_sc as plsc

L = 16
NC = 2
NS = 16
NW = NC * NS

ROWS = 16384
COLS = 200
D = 3
VOCAB = 10
RPW = ROWS // NW
CI = 128
NCH = RPW // CI
NG = CI // L
SLABS = ((0, 48), (48, 48), (96, 48), (144, 56))


def _make_kernel():
    mesh = plsc.VectorSubcoreMesh(core_axis_name="c", subcore_axis_name="s")

    @functools.partial(
        pl.kernel,
        out_type=jax.ShapeDtypeStruct((D, COLS, ROWS), jnp.float32),
        mesh=mesh,
        scratch_types=[
            [pltpu.VMEM((L,), jnp.float32) for _ in range(D)],
            [pltpu.VMEM((COLS, CI), jnp.int32) for _ in range(2)],
            [pltpu.VMEM((COLS, CI), jnp.float32) for _ in range(D)],
            [pltpu.SemaphoreType.DMA for _ in range(2)],
            [pltpu.SemaphoreType.DMA for _ in range(len(SLABS))],
            pltpu.SemaphoreType.DMA,
        ],
        compiler_params=pltpu.CompilerParams(needs_layout_passes=False),
    )
    def emb_kernel(xt_hbm, tab_hbm, out_hbm, tab_v, xb, ob, in_sem, out_sem,
                   tab_sem):
        wid = lax.axis_index("s") * NC + lax.axis_index("c")
        base = wid * RPW

        def in_slice(ch):
            return xt_hbm.at[:, pl.ds(base + ch * CI, CI)]

        def out_slab(ch, k, s):
            j0, jn = SLABS[s]
            return out_hbm.at[k, pl.ds(j0, jn), pl.ds(base + ch * CI, CI)]

        def ob_slab(k, s):
            j0, jn = SLABS[s]
            return ob[k].at[pl.ds(j0, jn), :]

        def compute_slab(b, s):
            j0, jn = SLABS[s]

            @plsc.parallel_loop(j0, j0 + jn, unroll=2)
            def row_loop(j):
                for g in range(NG):
                    sl = pl.ds(g * L, L)
                    idx = xb[b][j, sl]
                    for k in range(D):
                        vals = plsc.load_gather(tab_v[k], [idx])
                        ob[k][j, sl] = vals

        pltpu.async_copy(in_slice(0), xb[0], in_sem[0])
        if NCH > 1:
            pltpu.async_copy(in_slice(1), xb[1], in_sem[1])
        for k in range(D):
            pltpu.async_copy(tab_hbm.at[pl.ds(k * L, L)], tab_v[k], tab_sem)
        for k in range(D):
            pltpu.make_async_copy(
                tab_hbm.at[pl.ds(k * L, L)], tab_v[k], tab_sem
            ).wait()
        for ch in range(NCH):
            b = ch % 2
            pltpu.make_async_copy(in_slice(ch), xb[b], in_sem[b]).wait()
            for s in range(len(SLABS)):
                if ch >= 1:
                    for k in range(D):
                        pltpu.make_async_copy(
                            ob_slab(k, s), out_slab(ch - 1, k, s), out_sem[s]
                        ).wait()
                compute_slab(b, s)
                for k in range(D):
                    pltpu.async_copy(ob_slab(k, s), out_slab(ch, k, s), out_sem[s])
            if ch + 2 < NCH:
                pltpu.async_copy(in_slice(ch + 2), xb[b], in_sem[b])
        for s in range(len(SLABS)):
            for k in range(D):
                pltpu.make_async_copy(
                    ob_slab(k, s), out_slab(NCH - 1, k, s), out_sem[s]
                ).wait()

    return emb_kernel


_emb = _make_kernel()


def kernel(x, table):
    tab_p = (
        jnp.zeros((D, L), jnp.float32).at[:, :VOCAB].set(table.T).reshape(D * L)
    )
    out = _emb(x.T, tab_p)
    return out.transpose(2, 1, 0)
